# Initial kernel scaffold; baseline (speedup 1.0000x reference)
import jax
import jax.numpy as jnp
from jax.experimental import pallas as pl


def kernel(x, edge_index0, edge_index1, W0, b0, W1, b1):
    raise NotImplementedError("write your pallas kernel here")


# speedup vs baseline: 5.3477x; 5.3477x over previous
import functools

import jax
import jax.numpy as jnp
from jax import lax
from jax.experimental import pallas as pl
from jax.experimental.pallas import tpu as pltpu
from jax.experimental.pallas import tpu_sc as plsc

NC = 2
NS = 16
LANES = 16
CW = 16
CHUNK = 128


def _sc_aggregate(x_pad, src, dst, n_pad):
    n, dw = x_pad.shape
    d = dw - CW
    n_steps = src.shape[2]
    rows_per_tile = n_pad // NS

    mesh = plsc.VectorSubcoreMesh(
        core_axis_name="c", subcore_axis_name="s", num_cores=NC,
        num_subcores=NS)

    @functools.partial(
        pl.kernel,
        out_type=(
            jax.ShapeDtypeStruct((NC, n_pad, d), jnp.float32),
            jax.ShapeDtypeStruct((NC, n_pad, CW), jnp.float32),
        ),
        mesh=mesh,
        scratch_types=[
            pltpu.VMEM((n_steps, CHUNK), jnp.int32),
            pltpu.VMEM((n_steps, CHUNK), jnp.int32),
            pltpu.VMEM((CHUNK, dw), jnp.float32),
            pltpu.VMEM_SHARED((n_pad, dw), jnp.float32),
            pltpu.SemaphoreType.DMA,
        ],
        compiler_params=pltpu.CompilerParams(use_tc_tiling_on_sc=False),
    )
    def k(x_hbm, src_hbm, dst_hbm, agg_hbm, cnt_hbm,
          sidx_v, didx_v, rows_v, acc_sh, sem):
        c = lax.axis_index("c")
        s = lax.axis_index("s")
        row_base = s * rows_per_tile

        zero16 = jnp.zeros((LANES,), jnp.float32)

        def zrow(i, carry):
            def zcol(j, carry2):
                rows_v[i, pl.ds(j * LANES, LANES)] = zero16
                return carry2
            return lax.fori_loop(0, dw // LANES, zcol, carry)
        lax.fori_loop(0, CHUNK, zrow, 0)
        for r in range(rows_per_tile // CHUNK):
            pltpu.sync_copy(rows_v,
                            acc_sh.at[pl.ds(row_base + r * CHUNK, CHUNK)])

        pltpu.sync_copy(src_hbm.at[c, s], sidx_v)
        pltpu.sync_copy(dst_hbm.at[c, s], didx_v)
        plsc.subcore_barrier()

        def step(j, carry):
            pltpu.async_copy(x_hbm.at[sidx_v.at[j]], rows_v, sem).wait()
            pltpu.sync_copy(rows_v, acc_sh.at[didx_v.at[j]], add=True)
            return carry
        lax.fori_loop(0, n_steps, step, 0)

        plsc.subcore_barrier()
        pltpu.sync_copy(
            acc_sh.at[pl.ds(row_base, rows_per_tile), pl.ds(0, d)],
            agg_hbm.at[c, pl.ds(row_base, rows_per_tile)])
        pltpu.sync_copy(
            acc_sh.at[pl.ds(row_base, rows_per_tile), pl.ds(d, CW)],
            cnt_hbm.at[c, pl.ds(row_base, rows_per_tile)])

    return k(x_pad, src, dst)


def _combine(agg, cnt, W0, W1, b0, b1, n):
    d = agg.shape[2]
    blk = 2000
    grid = n // blk

    def body(agg_ref, cnt_ref, w0_ref, w1_ref, b0_ref, b1_ref, out_ref):
        dn = (((1,), (1,)), ((), ()))
        c0 = jnp.max(cnt_ref[0], axis=1, keepdims=True)
        c1 = jnp.max(cnt_ref[1], axis=1, keepdims=True)
        m0 = agg_ref[0] / jnp.maximum(c0, 1.0)
        m1 = agg_ref[1] / jnp.maximum(c1, 1.0)
        h = lax.dot_general(m0, w0_ref[...], dn,
                            preferred_element_type=jnp.float32)
        h = h + lax.dot_general(m1, w1_ref[...], dn,
                                preferred_element_type=jnp.float32)
        h = h + jnp.where(c0 > 0.0, 1.0, 0.0) * b0_ref[...]
        h = h + jnp.where(c1 > 0.0, 1.0, 0.0) * b1_ref[...]
        out_ref[...] = h

    return pl.pallas_call(
        body,
        grid=(grid,),
        in_specs=[
            pl.BlockSpec((2, blk, d), lambda i: (0, i, 0)),
            pl.BlockSpec((2, blk, CW), lambda i: (0, i, 0)),
            pl.BlockSpec((d, d), lambda i: (0, 0)),
            pl.BlockSpec((d, d), lambda i: (0, 0)),
            pl.BlockSpec((1, d), lambda i: (0, 0)),
            pl.BlockSpec((1, d), lambda i: (0, 0)),
        ],
        out_specs=pl.BlockSpec((blk, d), lambda i: (i, 0)),
        out_shape=jax.ShapeDtypeStruct((n, d), jnp.float32),
    )(agg, cnt, W0, W1, b0.reshape(1, d), b1.reshape(1, d))


def kernel(x, edge_index0, edge_index1, W0, b0, W1, b1):
    n, d = x.shape
    e = edge_index0.shape[1]

    e_pad = -(-e // (NS * CHUNK)) * (NS * CHUNK)
    n_steps = e_pad // (NS * CHUNK)
    n_pad = -(-(n + 1) // (NS * CHUNK)) * (NS * CHUNK)

    def prep(ei, row):
        idx = ei[row].astype(jnp.int32)
        pad_val = jnp.int32(0) if row == 0 else jnp.int32(n)
        idx = jnp.concatenate(
            [idx, jnp.full((e_pad - e,), pad_val, jnp.int32)])
        return idx.reshape(NS, n_steps, CHUNK)

    src = jnp.stack([prep(edge_index0, 0), prep(edge_index1, 0)])
    dst = jnp.stack([prep(edge_index0, 1), prep(edge_index1, 1)])

    x_pad = jnp.concatenate(
        [x.astype(jnp.float32), jnp.ones((n, CW), jnp.float32)], axis=1)

    with jax.enable_x64(False):
        agg, cnt = _sc_aggregate(x_pad, src, dst, n_pad)
        return _combine(agg, cnt, W0.astype(jnp.float32),
                        W1.astype(jnp.float32), b0.astype(jnp.float32),
                        b1.astype(jnp.float32), n)


# speedup vs baseline: 5.9839x; 1.1190x over previous
import functools

import jax
import jax.numpy as jnp
from jax import lax
from jax.experimental import pallas as pl
from jax.experimental.pallas import tpu as pltpu
from jax.experimental.pallas import tpu_sc as plsc

NC = 2
NS = 16
LANES = 16
CW = 16
CHUNK = 64
NBUF = 2


def _sc_aggregate(x_pad, src, dst, n_pad):
    n, dw = x_pad.shape
    d = dw - CW
    n_steps = src.shape[2]
    rows_per_tile = n_pad // NS

    mesh = plsc.VectorSubcoreMesh(
        core_axis_name="c", subcore_axis_name="s", num_cores=NC,
        num_subcores=NS)

    @functools.partial(
        pl.kernel,
        out_type=(
            jax.ShapeDtypeStruct((NC, n_pad, d), jnp.float32),
            jax.ShapeDtypeStruct((NC, n_pad, CW), jnp.float32),
        ),
        mesh=mesh,
        scratch_types=[
            pltpu.VMEM((n_steps, CHUNK), jnp.int32),
            pltpu.VMEM((n_steps, CHUNK), jnp.int32),
            pltpu.VMEM((NBUF, CHUNK, dw), jnp.float32),
            pltpu.VMEM_SHARED((n_pad, dw), jnp.float32),
            pltpu.SemaphoreType.DMA((NBUF,)),
            pltpu.SemaphoreType.DMA((NBUF,)),
        ],
        compiler_params=pltpu.CompilerParams(use_tc_tiling_on_sc=False),
    )
    def k(x_hbm, src_hbm, dst_hbm, agg_hbm, cnt_hbm,
          sidx_v, didx_v, rows_v, acc_sh, sem_g, sem_s):
        c = lax.axis_index("c")
        s = lax.axis_index("s")
        row_base = s * rows_per_tile

        zero16 = jnp.zeros((LANES,), jnp.float32)

        def zrow(i, carry):
            def zcol(j, carry2):
                rows_v[0, i, pl.ds(j * LANES, LANES)] = zero16
                return carry2
            return lax.fori_loop(0, dw // LANES, zcol, carry)
        lax.fori_loop(0, CHUNK, zrow, 0)
        for r in range(rows_per_tile // CHUNK):
            pltpu.sync_copy(rows_v.at[0],
                            acc_sh.at[pl.ds(row_base + r * CHUNK, CHUNK)])
        rem = rows_per_tile % CHUNK
        if rem:
            pltpu.sync_copy(
                rows_v.at[0, pl.ds(0, rem)],
                acc_sh.at[pl.ds(row_base + rows_per_tile - rem, rem)])

        pltpu.sync_copy(src_hbm.at[c, s], sidx_v)
        pltpu.sync_copy(dst_hbm.at[c, s], didx_v)
        plsc.subcore_barrier()

        n_groups = n_steps // NBUF

        def gather_start(j, b):
            pltpu.async_copy(x_hbm.at[sidx_v.at[j]], rows_v.at[b],
                             sem_g.at[b])

        def gather_wait(j, b):
            pltpu.make_async_copy(x_hbm.at[pl.ds(0, CHUNK)], rows_v.at[b],
                                  sem_g.at[b]).wait()

        def scatter(j, b):
            pltpu.sync_copy(rows_v.at[b], acc_sh.at[didx_v.at[j]], add=True)

        for b in range(NBUF):
            gather_start(b, b)

        def group(g, carry):
            base = g * NBUF
            for b in range(NBUF):
                gather_wait(base + b, b)
                scatter(base + b, b)
                gather_start(base + NBUF + b, b)
            return carry
        lax.fori_loop(0, n_groups - 1, group, 0)

        base = (n_groups - 1) * NBUF
        for b in range(NBUF):
            gather_wait(base + b, b)
            scatter(base + b, b)

        plsc.subcore_barrier()
        pltpu.sync_copy(
            acc_sh.at[pl.ds(row_base, rows_per_tile), pl.ds(0, d)],
            agg_hbm.at[c, pl.ds(row_base, rows_per_tile)])
        pltpu.sync_copy(
            acc_sh.at[pl.ds(row_base, rows_per_tile), pl.ds(d, CW)],
            cnt_hbm.at[c, pl.ds(row_base, rows_per_tile)])

    return k(x_pad, src, dst)


def _combine(agg, cnt, W0, W1, b0, b1, n):
    d = agg.shape[2]
    blk = 2000
    grid = n // blk

    def body(agg_ref, cnt_ref, w0_ref, w1_ref, b0_ref, b1_ref, out_ref):
        dn = (((1,), (1,)), ((), ()))
        c0 = jnp.max(cnt_ref[0], axis=1, keepdims=True)
        c1 = jnp.max(cnt_ref[1], axis=1, keepdims=True)
        m0 = agg_ref[0] / jnp.maximum(c0, 1.0)
        m1 = agg_ref[1] / jnp.maximum(c1, 1.0)
        h = lax.dot_general(m0, w0_ref[...], dn,
                            preferred_element_type=jnp.float32)
        h = h + lax.dot_general(m1, w1_ref[...], dn,
                                preferred_element_type=jnp.float32)
        h = h + jnp.where(c0 > 0.0, 1.0, 0.0) * b0_ref[...]
        h = h + jnp.where(c1 > 0.0, 1.0, 0.0) * b1_ref[...]
        out_ref[...] = h

    return pl.pallas_call(
        body,
        grid=(grid,),
        in_specs=[
            pl.BlockSpec((2, blk, d), lambda i: (0, i, 0)),
            pl.BlockSpec((2, blk, CW), lambda i: (0, i, 0)),
            pl.BlockSpec((d, d), lambda i: (0, 0)),
            pl.BlockSpec((d, d), lambda i: (0, 0)),
            pl.BlockSpec((1, d), lambda i: (0, 0)),
            pl.BlockSpec((1, d), lambda i: (0, 0)),
        ],
        out_specs=pl.BlockSpec((blk, d), lambda i: (i, 0)),
        out_shape=jax.ShapeDtypeStruct((n, d), jnp.float32),
    )(agg, cnt, W0, W1, b0.reshape(1, d), b1.reshape(1, d))


def kernel(x, edge_index0, edge_index1, W0, b0, W1, b1):
    n, d = x.shape
    e = edge_index0.shape[1]

    e_pad = -(-e // (NS * CHUNK * NBUF)) * (NS * CHUNK * NBUF)
    n_steps = e_pad // (NS * CHUNK)
    n_pad = -(-(n + 1) // (NS * 8)) * (NS * 8)

    def prep(ei, row):
        idx = ei[row].astype(jnp.int32)
        pad_val = jnp.int32(0) if row == 0 else jnp.int32(n)
        idx = jnp.concatenate(
            [idx, jnp.full((e_pad - e,), pad_val, jnp.int32)])
        return idx.reshape(NS, n_steps, CHUNK)

    src = jnp.stack([prep(edge_index0, 0), prep(edge_index1, 0)])
    dst = jnp.stack([prep(edge_index0, 1), prep(edge_index1, 1)])

    x_pad = jnp.concatenate(
        [x.astype(jnp.float32), jnp.ones((n, CW), jnp.float32)], axis=1)

    with jax.enable_x64(False):
        agg, cnt = _sc_aggregate(x_pad, src, dst, n_pad)
        return _combine(agg, cnt, W0.astype(jnp.float32),
                        W1.astype(jnp.float32), b0.astype(jnp.float32),
                        b1.astype(jnp.float32), n)
